# Initial kernel scaffold; baseline (speedup 1.0000x reference)
import jax
import jax.numpy as jnp
from jax.experimental import pallas as pl


def kernel(surface_xyz, surface_normals, features, Wq1, bq1, Wq2, bq2, Win1, bin1, Win2, bin2, g_in, be_in, A1, B1, A2, B2, Wout1, bout1, Wout2, bout2, g_out, be_out, Wl1, bl1, Wl2, bl2, Wt, bt):
    raise NotImplementedError("write your pallas kernel here")


# speedup vs baseline: 5.5002x; 5.5002x over previous
import jax
import jax.numpy as jnp
import numpy as np
from jax.experimental import pallas as pl
from jax.experimental.pallas import tpu as pltpu

_N = 3000
_NPAD = 3072
_BI = 256
_H = 16
_CUTS = 8
_GROUPS = 4
_EPS = 1e-5
_RADIUS = 9.0
_SENT = 1e4
_INV_SQRT2 = 0.7071067811865476


def _lrelu(v):
    return jnp.where(v >= 0, v, 0.2 * v)


def _masked_group_norm(z, gamma, beta):
    row = jax.lax.broadcasted_iota(jnp.int32, (_NPAD, 1), 0)
    mask = (row < _N).astype(jnp.float32)
    zm = z * mask
    cnt = float(_N * (_H // _GROUPS))
    cols = _H // _GROUPS
    parts = []
    for g in range(_GROUPS):
        sl = zm[:, g * cols:(g + 1) * cols]
        m = jnp.sum(sl) / cnt
        v = jnp.sum(((sl - m) * mask) ** 2) / cnt
        xn = (z[:, g * cols:(g + 1) * cols] - m) * jax.lax.rsqrt(v + _EPS)
        parts.append(xn)
    return jnp.concatenate(parts, axis=1) * gamma + beta


def _prologue_kernel(feat_ref, wq1t_ref, bq1_ref, wq2t_ref, bq2_ref,
                     win1t_ref, bin1_ref, win2t_ref, bin2_ref,
                     gin_ref, bein_ref, a2t_ref, b2_ref,
                     w_ref, ff_ref):
    feat = feat_ref[...]
    h1 = _lrelu(jnp.dot(feat, wq1t_ref[...],
                        preferred_element_type=jnp.float32) + bq1_ref[...])
    w_ref[...] = jnp.dot(h1, wq2t_ref[...],
                         preferred_element_type=jnp.float32) + bq2_ref[...]
    z = _lrelu(jnp.dot(feat, win1t_ref[...],
                       preferred_element_type=jnp.float32) + bin1_ref[...])
    z = _lrelu(jnp.dot(z, win2t_ref[...],
                       preferred_element_type=jnp.float32) + bin2_ref[...])
    f = _masked_group_norm(z, gin_ref[...], bein_ref[...])
    for k in range(_CUTS):
        ff_ref[k * _NPAD:(k + 1) * _NPAD, :] = f * a2t_ref[k:k + 1, :]
    ff_ref[_CUTS * _NPAD:(_CUTS + 1) * _NPAD, :] = f * b2_ref[...]


def _main_kernel(pn_ref, rt_ref, p48_ref, ff_ref, a1b_ref, xi_ref):
    blk = pn_ref[...]
    pxi, pyi, pzi = blk[:, 0:1], blk[:, 1:2], blk[:, 2:3]
    nxi, nyi, nzi = blk[:, 3:4], blk[:, 4:5], blk[:, 5:6]
    pxr = rt_ref[0:1, :]
    pyr = rt_ref[1:2, :]
    pzr = rt_ref[2:3, :]
    nxr = rt_ref[3:4, :]
    nyr = rt_ref[4:5, :]
    nzr = rt_ref[5:6, :]
    wr = rt_ref[6:7, :]

    dx = pxr - pxi
    dy = pyr - pyi
    dz = pzr - pzi
    q = 2.0 - (nxi * nxr + nyi * nyr + nzi * nzr)
    d2 = (dx * dx + dy * dy + dz * dz) * (q * q)
    e = jnp.exp(-0.5 * d2)
    win1 = e * e * wr

    s4 = jnp.dot(win1, p48_ref[...], preferred_element_type=jnp.float32)
    ss = s4[:, 3:4]
    gx = s4[:, 0:1] - ss * pxi
    gy = s4[:, 1:2] - ss * pyi
    gz = s4[:, 2:3] - ss * pzi

    s = 2.0 * (nzi >= 0).astype(jnp.float32) - 1.0
    a = -1.0 / (s + nzi)
    b = nxi * nyi * a
    u0 = 1.0 + s * nxi * nxi * a
    u1 = s * b
    u2 = -s * nxi
    v0 = b
    v1 = s + nyi * nyi * a
    v2 = -nyi

    ex = u0 * gx + u1 * gy + u2 * gz + 1e-5
    ey = v0 * gx + v1 * gy + v2 * gz + 1e-5
    inv = 1.0 / jnp.maximum(jnp.sqrt(ex * ex + ey * ey), 1e-12)
    ex = ex * inv
    ey = ey * inv

    c = _INV_SQRT2
    b00, b01, b02 = nxi * c, nyi * c, nzi * c
    b10 = (ex * u0 + ey * v0) * c
    b11 = (ex * u1 + ey * v1) * c
    b12 = (ex * u2 + ey * v2) * c
    b20 = (ex * v0 - ey * u0) * c
    b21 = (ex * v1 - ey * u1) * c
    b22 = (ex * v2 - ey * u2) * c

    d0 = b00 * pxr + b01 * pyr + b02 * pzr
    d1 = b10 * pxr + b11 * pyr + b12 * pzr
    d2p = b20 * pxr + b21 * pyr + b22 * pzr
    e0 = b00 * pxi + b01 * pyi + b02 * pzi
    e1 = b10 * pxi + b11 * pyi + b12 * pzi
    e2 = b20 * pxi + b21 * pyi + b22 * pzi

    acc = jnp.dot(e, ff_ref[_CUTS * _NPAD:(_CUTS + 1) * _NPAD, :],
                  preferred_element_type=jnp.float32)
    for k in range(_CUTS):
        a0 = a1b_ref[k, 0]
        a1 = a1b_ref[k, 1]
        a2 = a1b_ref[k, 2]
        bk = a1b_ref[k, 3]
        rowk = a0 * d0 + a1 * d1 + a2 * d2p
        biask = bk - (a0 * e0 + a1 * e1 + a2 * e2)
        g = jnp.maximum(rowk + biask, 0.0) * e
        acc = acc + jnp.dot(g, ff_ref[k * _NPAD:(k + 1) * _NPAD, :],
                            preferred_element_type=jnp.float32)
    xi_ref[...] = acc


def _epilogue_kernel(xi_ref, feat_ref,
                     wout1t_ref, bout1_ref, wout2t_ref, bout2_ref,
                     gout_ref, beout_ref,
                     wl1t_ref, bl1_ref, wl2t_ref, bl2_ref,
                     wtt_ref, bt_ref, out_ref):
    t = _lrelu(jnp.dot(xi_ref[...], wout1t_ref[...],
                       preferred_element_type=jnp.float32) + bout1_ref[...])
    t = _lrelu(jnp.dot(t, wout2t_ref[...],
                       preferred_element_type=jnp.float32) + bout2_ref[...])
    t = _masked_group_norm(t, gout_ref[...], beout_ref[...])
    t = jnp.dot(jnp.maximum(jnp.dot(t, wl1t_ref[...],
                                    preferred_element_type=jnp.float32)
                            + bl1_ref[...], 0.0),
                wl2t_ref[...], preferred_element_type=jnp.float32) + bl2_ref[...]
    out_ref[...] = t + jnp.dot(feat_ref[...], wtt_ref[...],
                               preferred_element_type=jnp.float32) + bt_ref[...]


def kernel(surface_xyz, surface_normals, features, Wq1, bq1, Wq2, bq2,
           Win1, bin1, Win2, bin2, g_in, be_in, A1, B1, A2, B2,
           Wout1, bout1, Wout2, bout2, g_out, be_out,
           Wl1, bl1, Wl2, bl2, Wt, bt):
    pad = _NPAD - _N
    pts = surface_xyz / _RADIUS
    pts_p = jnp.concatenate(
        [pts, jnp.full((pad, 3), _SENT, jnp.float32)], axis=0)
    nrm_p = jnp.pad(surface_normals, ((0, pad), (0, 0)))
    feat_p = jnp.pad(features, ((0, pad), (0, 0)))

    row1 = lambda v: v.reshape(1, -1)

    w_p, ff = pl.pallas_call(
        _prologue_kernel,
        out_shape=[
            jax.ShapeDtypeStruct((_NPAD, 1), jnp.float32),
            jax.ShapeDtypeStruct(((_CUTS + 1) * _NPAD, _H), jnp.float32),
        ],
    )(feat_p, Wq1.T, row1(bq1), Wq2.T, row1(bq2),
      Win1.T, row1(bin1), Win2.T, row1(bin2),
      row1(g_in), row1(be_in), A2.T, row1(B2))

    zcol = jnp.zeros((_NPAD, 1), jnp.float32)
    pn = jnp.concatenate([pts_p, nrm_p, zcol, zcol], axis=1)
    rt = jnp.concatenate([pts_p, nrm_p, w_p, zcol], axis=1).T
    p48 = jnp.concatenate(
        [pts_p, jnp.ones((_NPAD, 1), jnp.float32),
         jnp.zeros((_NPAD, 4), jnp.float32)], axis=1)
    a1b = jnp.concatenate([A1, B1.reshape(-1, 1)], axis=1)

    xi = pl.pallas_call(
        _main_kernel,
        grid=(_NPAD // _BI,),
        in_specs=[
            pl.BlockSpec((_BI, 8), lambda i: (i, 0)),
            pl.BlockSpec((8, _NPAD), lambda i: (0, 0)),
            pl.BlockSpec((_NPAD, 8), lambda i: (0, 0)),
            pl.BlockSpec(((_CUTS + 1) * _NPAD, _H), lambda i: (0, 0)),
            pl.BlockSpec(memory_space=pltpu.SMEM),
        ],
        out_specs=pl.BlockSpec((_BI, _H), lambda i: (i, 0)),
        out_shape=jax.ShapeDtypeStruct((_NPAD, _H), jnp.float32),
    )(pn, rt, p48, ff, a1b)

    out = pl.pallas_call(
        _epilogue_kernel,
        out_shape=jax.ShapeDtypeStruct((_NPAD, _H), jnp.float32),
    )(xi, feat_p, Wout1.T, row1(bout1), Wout2.T, row1(bout2),
      row1(g_out), row1(be_out), Wl1.T, row1(bl1), Wl2.T, row1(bl2),
      Wt.T, row1(bt))

    return out[:_N, :]


# speedup vs baseline: 6.0540x; 1.1007x over previous
import jax
import jax.numpy as jnp
import numpy as np
from jax.experimental import pallas as pl
from jax.experimental.pallas import tpu as pltpu

_N = 3000
_NPAD = 3072
_BI = 256
_H = 16
_CUTS = 8
_GROUPS = 4
_EPS = 1e-5
_RADIUS = 9.0
_SENT = 1e4
_INV_SQRT2 = 0.7071067811865476


def _lrelu(v):
    return jnp.where(v >= 0, v, 0.2 * v)


def _masked_group_norm(z, gamma, beta):
    row = jax.lax.broadcasted_iota(jnp.int32, (_NPAD, 1), 0)
    mask = (row < _N).astype(jnp.float32)
    zm = z * mask
    cnt = float(_N * (_H // _GROUPS))
    cols = _H // _GROUPS
    parts = []
    for g in range(_GROUPS):
        sl = zm[:, g * cols:(g + 1) * cols]
        m = jnp.sum(sl) / cnt
        v = jnp.sum(((sl - m) * mask) ** 2) / cnt
        xn = (z[:, g * cols:(g + 1) * cols] - m) * jax.lax.rsqrt(v + _EPS)
        parts.append(xn)
    return jnp.concatenate(parts, axis=1) * gamma + beta


def _prologue_kernel(feat_ref, wq1t_ref, bq1_ref, wq2t_ref, bq2_ref,
                     win1t_ref, bin1_ref, win2t_ref, bin2_ref,
                     gin_ref, bein_ref, a2t_ref, b2_ref,
                     w_ref, ff_ref):
    feat = feat_ref[...]
    h1 = _lrelu(jnp.dot(feat, wq1t_ref[...],
                        preferred_element_type=jnp.float32) + bq1_ref[...])
    w_ref[...] = jnp.dot(h1, wq2t_ref[...],
                         preferred_element_type=jnp.float32) + bq2_ref[...]
    z = _lrelu(jnp.dot(feat, win1t_ref[...],
                       preferred_element_type=jnp.float32) + bin1_ref[...])
    z = _lrelu(jnp.dot(z, win2t_ref[...],
                       preferred_element_type=jnp.float32) + bin2_ref[...])
    f = _masked_group_norm(z, gin_ref[...], bein_ref[...])
    for k in range(_CUTS):
        ff_ref[k * _NPAD:(k + 1) * _NPAD, :] = (
            f * a2t_ref[k:k + 1, :]).astype(jnp.bfloat16)
    ff_ref[_CUTS * _NPAD:(_CUTS + 1) * _NPAD, :] = (
        f * b2_ref[...]).astype(jnp.bfloat16)


def _main_kernel(pn_ref, rj8_ref, rj4b_ref, wr_ref, p48_ref, ff_ref, tall_ref,
                 xi_ref):
    blk = pn_ref[...]
    pxi, pyi, pzi = blk[:, 0:1], blk[:, 1:2], blk[:, 2:3]
    nxi, nyi, nzi = blk[:, 3:4], blk[:, 4:5], blk[:, 5:6]
    wr = wr_ref[...]

    pp = pxi * pxi + pyi * pyi + pzi * pzi
    zero = jnp.zeros_like(pxi)
    half = jnp.full_like(pxi, -0.5)
    two = jnp.full_like(pxi, 2.0)
    lhs_a = jnp.concatenate(
        [pxi, pyi, pzi, -0.5 * pp, half, zero, zero, zero], axis=1)
    lhs_b = jnp.concatenate(
        [zero, zero, zero, two, zero, -nxi, -nyi, -nzi], axis=1)
    lhs = jnp.concatenate([lhs_a, lhs_b], axis=0)
    zq = jnp.dot(lhs, rj8_ref[...], preferred_element_type=jnp.float32)
    z = zq[:_BI, :]
    q = zq[_BI:, :]
    e = jnp.exp(z * q * q)
    win1 = e * e * wr

    s4 = jnp.dot(win1, p48_ref[...], preferred_element_type=jnp.float32)
    ss = s4[:, 3:4]
    gx = s4[:, 0:1] - ss * pxi
    gy = s4[:, 1:2] - ss * pyi
    gz = s4[:, 2:3] - ss * pzi

    s = 2.0 * (nzi >= 0).astype(jnp.float32) - 1.0
    a = -1.0 / (s + nzi)
    b = nxi * nyi * a
    u0 = 1.0 + s * nxi * nxi * a
    u1 = s * b
    u2 = -s * nxi
    v0 = b
    v1 = s + nyi * nyi * a
    v2 = -nyi

    ex = u0 * gx + u1 * gy + u2 * gz + 1e-5
    ey = v0 * gx + v1 * gy + v2 * gz + 1e-5
    inv = 1.0 / jnp.maximum(jnp.sqrt(ex * ex + ey * ey), 1e-12)
    ex = ex * inv
    ey = ey * inv

    c = _INV_SQRT2
    b00, b01, b02 = nxi * c, nyi * c, nzi * c
    b10 = (ex * u0 + ey * v0) * c
    b11 = (ex * u1 + ey * v1) * c
    b12 = (ex * u2 + ey * v2) * c
    b20 = (ex * v0 - ey * u0) * c
    b21 = (ex * v1 - ey * u1) * c
    b22 = (ex * v2 - ey * u2) * c

    e0 = b00 * pxi + b01 * pyi + b02 * pzi
    e1 = b10 * pxi + b11 * pyi + b12 * pzi
    e2 = b20 * pxi + b21 * pyi + b22 * pzi

    ones = jnp.ones_like(pxi)
    basism = jnp.concatenate(
        [b00, b01, b02, b10, b11, b12, b20, b21, b22,
         -e0, -e1, -e2, ones], axis=1)
    lhs_all = jnp.dot(basism, tall_ref[...],
                      preferred_element_type=jnp.float32).astype(jnp.bfloat16)

    rj4 = rj4b_ref[...]
    acc = jnp.dot(e.astype(jnp.bfloat16),
                  ff_ref[_CUTS * _NPAD:(_CUTS + 1) * _NPAD, :],
                  preferred_element_type=jnp.float32)
    for k in range(_CUTS):
        rk = jnp.dot(lhs_all[:, 4 * k:4 * (k + 1)], rj4,
                     preferred_element_type=jnp.float32)
        g = (jnp.maximum(rk, 0.0) * e).astype(jnp.bfloat16)
        acc = acc + jnp.dot(g, ff_ref[k * _NPAD:(k + 1) * _NPAD, :],
                            preferred_element_type=jnp.float32)
    xi_ref[...] = acc


def _epilogue_kernel(xi_ref, feat_ref,
                     wout1t_ref, bout1_ref, wout2t_ref, bout2_ref,
                     gout_ref, beout_ref,
                     wl1t_ref, bl1_ref, wl2t_ref, bl2_ref,
                     wtt_ref, bt_ref, out_ref):
    t = _lrelu(jnp.dot(xi_ref[...], wout1t_ref[...],
                       preferred_element_type=jnp.float32) + bout1_ref[...])
    t = _lrelu(jnp.dot(t, wout2t_ref[...],
                       preferred_element_type=jnp.float32) + bout2_ref[...])
    t = _masked_group_norm(t, gout_ref[...], beout_ref[...])
    t = jnp.dot(jnp.maximum(jnp.dot(t, wl1t_ref[...],
                                    preferred_element_type=jnp.float32)
                            + bl1_ref[...], 0.0),
                wl2t_ref[...], preferred_element_type=jnp.float32) + bl2_ref[...]
    out_ref[...] = t + jnp.dot(feat_ref[...], wtt_ref[...],
                               preferred_element_type=jnp.float32) + bt_ref[...]


def kernel(surface_xyz, surface_normals, features, Wq1, bq1, Wq2, bq2,
           Win1, bin1, Win2, bin2, g_in, be_in, A1, B1, A2, B2,
           Wout1, bout1, Wout2, bout2, g_out, be_out,
           Wl1, bl1, Wl2, bl2, Wt, bt):
    pad = _NPAD - _N
    pts = surface_xyz / _RADIUS
    pts_p = jnp.concatenate(
        [pts, jnp.full((pad, 3), _SENT, jnp.float32)], axis=0)
    nrm_p = jnp.pad(surface_normals, ((0, pad), (0, 0)))
    feat_p = jnp.pad(features, ((0, pad), (0, 0)))

    row1 = lambda v: v.reshape(1, -1)

    w_p, ff = pl.pallas_call(
        _prologue_kernel,
        out_shape=[
            jax.ShapeDtypeStruct((_NPAD, 1), jnp.float32),
            jax.ShapeDtypeStruct(((_CUTS + 1) * _NPAD, _H), jnp.bfloat16),
        ],
    )(feat_p, Wq1.T, row1(bq1), Wq2.T, row1(bq2),
      Win1.T, row1(bin1), Win2.T, row1(bin2),
      row1(g_in), row1(be_in), A2.T, row1(B2))

    zcol = jnp.zeros((_NPAD, 1), jnp.float32)
    ocol = jnp.ones((_NPAD, 1), jnp.float32)
    pp_col = jnp.sum(pts_p * pts_p, axis=1, keepdims=True)
    pn = jnp.concatenate([pts_p, nrm_p, zcol, zcol], axis=1)
    rj8 = jnp.concatenate([pts_p, ocol, pp_col, nrm_p], axis=1).T
    rj4b = rj8[0:4, :].astype(jnp.bfloat16)
    p48 = jnp.concatenate(
        [pts_p, ocol, jnp.zeros((_NPAD, 4), jnp.float32)], axis=1)

    eye3 = jnp.eye(3, dtype=jnp.float32)
    tcols = []
    for k in range(_CUTS):
        top = jnp.concatenate(
            [jnp.kron(A1[k].reshape(3, 1), eye3),
             jnp.zeros((9, 1), jnp.float32)], axis=1)
        mid = jnp.concatenate(
            [jnp.zeros((3, 3), jnp.float32), A1[k].reshape(3, 1)], axis=1)
        bot = jnp.concatenate(
            [jnp.zeros((1, 3), jnp.float32), B1[k].reshape(1, 1)], axis=1)
        tcols.append(jnp.concatenate([top, mid, bot], axis=0))
    tall = jnp.concatenate(tcols, axis=1)

    xi = pl.pallas_call(
        _main_kernel,
        grid=(_NPAD // _BI,),
        in_specs=[
            pl.BlockSpec((_BI, 8), lambda i: (i, 0)),
            pl.BlockSpec((8, _NPAD), lambda i: (0, 0)),
            pl.BlockSpec((4, _NPAD), lambda i: (0, 0)),
            pl.BlockSpec((1, _NPAD), lambda i: (0, 0)),
            pl.BlockSpec((_NPAD, 8), lambda i: (0, 0)),
            pl.BlockSpec(((_CUTS + 1) * _NPAD, _H), lambda i: (0, 0)),
            pl.BlockSpec((13, 32), lambda i: (0, 0)),
        ],
        out_specs=pl.BlockSpec((_BI, _H), lambda i: (i, 0)),
        out_shape=jax.ShapeDtypeStruct((_NPAD, _H), jnp.float32),
    )(pn, rj8, rj4b, w_p.T, p48, ff, tall)

    out = pl.pallas_call(
        _epilogue_kernel,
        out_shape=jax.ShapeDtypeStruct((_NPAD, _H), jnp.float32),
    )(xi, feat_p, Wout1.T, row1(bout1), Wout2.T, row1(bout2),
      row1(g_out), row1(be_out), Wl1.T, row1(bl1), Wl2.T, row1(bl2),
      Wt.T, row1(bt))

    return out[:_N, :]


# speedup vs baseline: 6.3659x; 1.0515x over previous
import jax
import jax.numpy as jnp
from jax.experimental import pallas as pl
from jax.experimental.pallas import tpu as pltpu

_N = 3000
_NPAD = 3072
_BI = 256
_NBLK = _NPAD // _BI
_H = 16
_CUTS = 8
_GROUPS = 4
_EPS = 1e-5
_RADIUS = 9.0
_SENT = 8192.0
_INV_SQRT2 = 0.7071067811865476


def _lrelu(v):
    return jnp.where(v >= 0, v, 0.2 * v)


def _masked_group_norm(z, gamma, beta):
    row = jax.lax.broadcasted_iota(jnp.int32, (_NPAD, 1), 0)
    mask = (row < _N).astype(jnp.float32)
    zm = z * mask
    cnt = float(_N * (_H // _GROUPS))
    cols = _H // _GROUPS
    parts = []
    for g in range(_GROUPS):
        sl = zm[:, g * cols:(g + 1) * cols]
        m = jnp.sum(sl) / cnt
        v = jnp.sum(((sl - m) * mask) ** 2) / cnt
        xn = (z[:, g * cols:(g + 1) * cols] - m) * jax.lax.rsqrt(v + _EPS)
        parts.append(xn)
    return jnp.concatenate(parts, axis=1) * gamma + beta


def _fused_kernel(pn_ref, featt_ref, feat_ref, rj8b_ref, rj4b_ref, p48_ref,
                  tall_ref,
                  wq1_ref, bq1c_ref, wq2_ref, bq2c_ref,
                  win1t_ref, bin1_ref, win2t_ref, bin2_ref, gin_ref, bein_ref,
                  a2t_ref, b2_ref,
                  wout1t_ref, bout1_ref, wout2t_ref, bout2_ref,
                  gout_ref, beout_ref,
                  wl1t_ref, bl1_ref, wl2t_ref, bl2_ref, wtt_ref, bt_ref,
                  out_ref, wr_s, ff_s, xi_s):
    i = pl.program_id(0)

    @pl.when(i == 0)
    def _prologue():
        h1t = _lrelu(jnp.dot(wq1_ref[...], featt_ref[...],
                             preferred_element_type=jnp.float32)
                     + bq1c_ref[...])
        wr_s[...] = jnp.dot(wq2_ref[...], h1t,
                            preferred_element_type=jnp.float32) + bq2c_ref[...]
        feat = feat_ref[...]
        z = _lrelu(jnp.dot(feat, win1t_ref[...],
                           preferred_element_type=jnp.float32) + bin1_ref[...])
        z = _lrelu(jnp.dot(z, win2t_ref[...],
                           preferred_element_type=jnp.float32) + bin2_ref[...])
        f = _masked_group_norm(z, gin_ref[...], bein_ref[...])
        for k in range(_CUTS):
            ff_s[k * _NPAD:(k + 1) * _NPAD, :] = (
                f * a2t_ref[k:k + 1, :]).astype(jnp.bfloat16)
        ff_s[_CUTS * _NPAD:(_CUTS + 1) * _NPAD, :] = (
            f * b2_ref[...]).astype(jnp.bfloat16)

    @pl.when((i >= 1) & (i <= _NBLK))
    def _main():
        off = (i - 1) * _BI
        blk = pn_ref[pl.ds(off, _BI), :]
        pxi, pyi, pzi = blk[:, 0:1], blk[:, 1:2], blk[:, 2:3]
        nxi, nyi, nzi = blk[:, 3:4], blk[:, 4:5], blk[:, 5:6]
        wr = wr_s[...]

        pp = pxi * pxi + pyi * pyi + pzi * pzi
        zero = jnp.zeros_like(pxi)
        half = jnp.full_like(pxi, -0.5)
        two = jnp.full_like(pxi, 2.0)
        lhs_a = jnp.concatenate(
            [pxi, pyi, pzi, -0.5 * pp, half, zero, zero, zero], axis=1)
        lhs_b = jnp.concatenate(
            [zero, zero, zero, two, zero, -nxi, -nyi, -nzi], axis=1)
        lhs = jnp.concatenate([lhs_a, lhs_b], axis=0).astype(jnp.bfloat16)
        zq = jnp.dot(lhs, rj8b_ref[...], preferred_element_type=jnp.float32)
        z = zq[:_BI, :]
        q = zq[_BI:, :]
        e = jnp.exp(z * q * q)
        win1 = e * e * wr

        s4 = jnp.dot(win1, p48_ref[...], preferred_element_type=jnp.float32)
        ss = s4[:, 3:4]
        gx = s4[:, 0:1] - ss * pxi
        gy = s4[:, 1:2] - ss * pyi
        gz = s4[:, 2:3] - ss * pzi

        s = 2.0 * (nzi >= 0).astype(jnp.float32) - 1.0
        a = -1.0 / (s + nzi)
        b = nxi * nyi * a
        u0 = 1.0 + s * nxi * nxi * a
        u1 = s * b
        u2 = -s * nxi
        v0 = b
        v1 = s + nyi * nyi * a
        v2 = -nyi

        ex = u0 * gx + u1 * gy + u2 * gz + 1e-5
        ey = v0 * gx + v1 * gy + v2 * gz + 1e-5
        inv = 1.0 / jnp.maximum(jnp.sqrt(ex * ex + ey * ey), 1e-12)
        ex = ex * inv
        ey = ey * inv

        c = _INV_SQRT2
        b00, b01, b02 = nxi * c, nyi * c, nzi * c
        b10 = (ex * u0 + ey * v0) * c
        b11 = (ex * u1 + ey * v1) * c
        b12 = (ex * u2 + ey * v2) * c
        b20 = (ex * v0 - ey * u0) * c
        b21 = (ex * v1 - ey * u1) * c
        b22 = (ex * v2 - ey * u2) * c

        e0 = b00 * pxi + b01 * pyi + b02 * pzi
        e1 = b10 * pxi + b11 * pyi + b12 * pzi
        e2 = b20 * pxi + b21 * pyi + b22 * pzi

        ones = jnp.ones_like(pxi)
        basism = jnp.concatenate(
            [b00, b01, b02, b10, b11, b12, b20, b21, b22,
             -e0, -e1, -e2, ones], axis=1)
        lhs_all = jnp.dot(
            basism, tall_ref[...],
            preferred_element_type=jnp.float32).astype(jnp.bfloat16)

        rj4 = rj4b_ref[...]
        acc = jnp.dot(e.astype(jnp.bfloat16),
                      ff_s[_CUTS * _NPAD:(_CUTS + 1) * _NPAD, :],
                      preferred_element_type=jnp.float32)
        for k in range(_CUTS):
            rk = jnp.dot(lhs_all[:, 4 * k:4 * (k + 1)], rj4,
                         preferred_element_type=jnp.float32)
            g = (jnp.maximum(rk, 0.0) * e).astype(jnp.bfloat16)
            acc = acc + jnp.dot(g, ff_s[k * _NPAD:(k + 1) * _NPAD, :],
                                preferred_element_type=jnp.float32)
        xi_s[pl.ds(off, _BI), :] = acc

    @pl.when(i == _NBLK + 1)
    def _epilogue():
        t = _lrelu(jnp.dot(xi_s[...], wout1t_ref[...],
                           preferred_element_type=jnp.float32)
                   + bout1_ref[...])
        t = _lrelu(jnp.dot(t, wout2t_ref[...],
                           preferred_element_type=jnp.float32)
                   + bout2_ref[...])
        t = _masked_group_norm(t, gout_ref[...], beout_ref[...])
        t = jnp.dot(jnp.maximum(jnp.dot(t, wl1t_ref[...],
                                        preferred_element_type=jnp.float32)
                                + bl1_ref[...], 0.0),
                    wl2t_ref[...],
                    preferred_element_type=jnp.float32) + bl2_ref[...]
        out_ref[...] = t + jnp.dot(feat_ref[...], wtt_ref[...],
                                   preferred_element_type=jnp.float32
                                   ) + bt_ref[...]


def kernel(surface_xyz, surface_normals, features, Wq1, bq1, Wq2, bq2,
           Win1, bin1, Win2, bin2, g_in, be_in, A1, B1, A2, B2,
           Wout1, bout1, Wout2, bout2, g_out, be_out,
           Wl1, bl1, Wl2, bl2, Wt, bt):
    pad = _NPAD - _N
    pts = surface_xyz / _RADIUS
    pts_p = jnp.concatenate(
        [pts, jnp.full((pad, 3), _SENT, jnp.float32)], axis=0)
    nrm_p = jnp.pad(surface_normals, ((0, pad), (0, 0)))
    feat_p = jnp.pad(features, ((0, pad), (0, 0)))

    row1 = lambda v: v.reshape(1, -1)
    zcol = jnp.zeros((_NPAD, 1), jnp.float32)
    ocol = jnp.ones((_NPAD, 1), jnp.float32)
    pp_col = jnp.sum(pts_p * pts_p, axis=1, keepdims=True)
    pn = jnp.concatenate([pts_p, nrm_p, zcol, zcol], axis=1)
    rj8 = jnp.concatenate([pts_p, ocol, pp_col, nrm_p], axis=1).T
    rj8b = rj8.astype(jnp.bfloat16)
    rj4b = rj8b[0:4, :]
    p48 = jnp.concatenate(
        [pts_p, ocol, jnp.zeros((_NPAD, 4), jnp.float32)], axis=1)

    eye3 = jnp.eye(3, dtype=jnp.float32)
    tcols = []
    for k in range(_CUTS):
        top = jnp.concatenate(
            [jnp.kron(A1[k].reshape(3, 1), eye3),
             jnp.zeros((9, 1), jnp.float32)], axis=1)
        mid = jnp.concatenate(
            [jnp.zeros((3, 3), jnp.float32), A1[k].reshape(3, 1)], axis=1)
        bot = jnp.concatenate(
            [jnp.zeros((1, 3), jnp.float32), B1[k].reshape(1, 1)], axis=1)
        tcols.append(jnp.concatenate([top, mid, bot], axis=0))
    tall = jnp.concatenate(tcols, axis=1)

    const = lambda i: (0, 0)
    full = lambda shp: pl.BlockSpec(shp, const)
    out = pl.pallas_call(
        _fused_kernel,
        grid=(_NBLK + 2,),
        in_specs=[
            full((_NPAD, 8)),
            full((_H, _NPAD)),
            full((_NPAD, _H)),
            full((8, _NPAD)),
            full((4, _NPAD)),
            full((_NPAD, 8)),
            full((13, 32)),
            full((_H, _H)), full((_H, 1)), full((1, _H)), full((1, 1)),
            full((_H, _H)), full((1, _H)), full((_H, _H)), full((1, _H)),
            full((1, _H)), full((1, _H)),
            full((_CUTS, _H)), full((1, _H)),
            full((_H, _H)), full((1, _H)), full((_H, _H)), full((1, _H)),
            full((1, _H)), full((1, _H)),
            full((_H, _H)), full((1, _H)), full((_H, _H)), full((1, _H)),
            full((_H, _H)), full((1, _H)),
        ],
        out_specs=pl.BlockSpec((_NPAD, _H), const),
        out_shape=jax.ShapeDtypeStruct((_NPAD, _H), jnp.float32),
        scratch_shapes=[
            pltpu.VMEM((1, _NPAD), jnp.float32),
            pltpu.VMEM(((_CUTS + 1) * _NPAD, _H), jnp.bfloat16),
            pltpu.VMEM((_NPAD, _H), jnp.float32),
        ],
        compiler_params=pltpu.CompilerParams(
            dimension_semantics=("arbitrary",)),
    )(pn, feat_p.T, feat_p, rj8b, rj4b, p48, tall,
      Wq1, bq1.reshape(-1, 1), Wq2, bq2.reshape(1, 1),
      Win1.T, row1(bin1), Win2.T, row1(bin2), row1(g_in), row1(be_in),
      A2.T, row1(B2),
      Wout1.T, row1(bout1), Wout2.T, row1(bout2), row1(g_out), row1(be_out),
      Wl1.T, row1(bl1), Wl2.T, row1(bl2), Wt.T, row1(bt))

    return out[:_N, :]
